# baseline (speedup 1.0000x reference)
import functools

import jax
import jax.numpy as jnp
from jax import lax
from jax.experimental import pallas as pl
from jax.experimental.pallas import tpu as pltpu
from jax.experimental.pallas import tpu_sc as plsc

N = 10000
E = 160000
D = 256
H = 128
G = 64
EPS = 1e-5

NC = 2
NS = 16

CH = 125
NCHUNK = (E // NS) // CH
HCH = 40
WB = 640
WB_LAST = N - WB * (NS - 1)

CCH = 125
CROWS = (E // (NC * NS)) // CCH
CW = 128

@functools.cache
def _sc_kernels():
    mesh = plsc.VectorSubcoreMesh(core_axis_name="c", subcore_axis_name="s",
                                  num_cores=NC, num_subcores=NS)

    def shard_copy(src_ref, dst_ref, s, **kw):
        pl.when(s < NS - 1)(lambda: pltpu.sync_copy(
            src_ref.at[pl.ds(s * WB, WB)], dst_ref.at[pl.ds(s * WB, WB)], **kw))
        pl.when(s == NS - 1)(lambda: pltpu.sync_copy(
            src_ref.at[pl.ds((NS - 1) * WB, WB_LAST)],
            dst_ref.at[pl.ds((NS - 1) * WB, WB_LAST)], **kw))

    @functools.partial(
        pl.kernel,
        out_type=[jax.ShapeDtypeStruct((N, H), jnp.float32),
                  jax.ShapeDtypeStruct((N, H), jnp.float32)],
        mesh=mesh,
        scratch_types=[
            pltpu.VMEM((HCH, CH), jnp.int32),
            pltpu.VMEM((HCH, CH), jnp.int32),
            pltpu.VMEM((CH, H), jnp.float32),
            pltpu.VMEM((CH, H), jnp.float32),
            pltpu.VMEM_SHARED((N, H), jnp.float32),
            pltpu.SemaphoreType.DMA,
            pltpu.SemaphoreType.DMA,
        ],
    )
    def sc_aggregate(xlo_hbm, xhi_hbm, src_hbm, dst_hbm, zeros_hbm,
                     alo_hbm, ahi_hbm,
                     src_v, dst_v, rows_v0, rows_v1, acc_sh, sem0, sem1):
        c = lax.axis_index("c")
        s = lax.axis_index("s")
        shard_copy(zeros_hbm, acc_sh, s)
        plsc.subcore_barrier()

        def run(x_hbm):
            def g_start(j, buf, sem):
                pltpu.async_copy(x_hbm.at[src_v.at[j]], buf, sem)

            def g_wait(buf, sem):
                pltpu.make_async_copy(x_hbm.at[src_v.at[0]], buf, sem).wait()

            def window(hbase):
                pltpu.sync_copy(src_hbm.at[pl.ds(hbase, HCH)], src_v)
                pltpu.sync_copy(dst_hbm.at[pl.ds(hbase, HCH)], dst_v)
                g_start(0, rows_v0, sem0)
                g_start(1, rows_v1, sem1)

                def step(jj, carry):
                    j0 = jj * 2
                    j1 = j0 + 1
                    g_wait(rows_v0, sem0)
                    pltpu.sync_copy(rows_v0, acc_sh.at[dst_v.at[j0]],
                                    add=True)
                    pl.when(j0 + 2 < HCH)(
                        lambda: g_start(j0 + 2, rows_v0, sem0))
                    g_wait(rows_v1, sem1)
                    pltpu.sync_copy(rows_v1, acc_sh.at[dst_v.at[j1]],
                                    add=True)
                    pl.when(j1 + 2 < HCH)(
                        lambda: g_start(j1 + 2, rows_v1, sem1))
                    return carry
                lax.fori_loop(0, HCH // 2, step, 0)

            def hstep(hh, carry):
                window(s * NCHUNK + hh * HCH)
                return carry
            lax.fori_loop(0, NCHUNK // HCH, hstep, 0)

        pl.when(c == 0)(lambda: run(xlo_hbm))
        pl.when(c == 1)(lambda: run(xhi_hbm))
        plsc.subcore_barrier()

        pl.when(c == 0)(lambda: shard_copy(acc_sh, alo_hbm, s))
        pl.when(c == 1)(lambda: shard_copy(acc_sh, ahi_hbm, s))

    @functools.partial(
        pl.kernel,
        out_type=[jax.ShapeDtypeStruct((N, CW), jnp.float32),
                  jax.ShapeDtypeStruct((N, CW), jnp.float32)],
        mesh=mesh,
        scratch_types=[
            pltpu.VMEM((CROWS, CCH), jnp.int32),
            pltpu.VMEM((CCH, CW), jnp.float32),
            pltpu.VMEM_SHARED((N, CW), jnp.float32),
        ],
    )
    def sc_counts(dst_hbm, ones_hbm, zeros_hbm, cnta_hbm, cntb_hbm,
                  dst_v, ones_v, cnt_sh):
        c = lax.axis_index("c")
        s = lax.axis_index("s")
        w = s * NC + c
        shard_copy(zeros_hbm, cnt_sh, s)
        pltpu.sync_copy(ones_hbm, ones_v)
        pltpu.sync_copy(dst_hbm.at[pl.ds(w * CROWS, CROWS)], dst_v)
        plsc.subcore_barrier()

        def step(j, carry):
            pltpu.sync_copy(ones_v, cnt_sh.at[dst_v.at[j]], add=True)
            return carry
        lax.fori_loop(0, CROWS, step, 0)
        plsc.subcore_barrier()

        pl.when(c == 0)(lambda: shard_copy(cnt_sh, cnta_hbm, s))
        pl.when(c == 1)(lambda: shard_copy(cnt_sh, cntb_hbm, s))

    return sc_aggregate, sc_counts


RB = 1000
NBLK = N // RB


def _tu_body(hlo, hhi, wr, b, u_ref):
    h = jnp.concatenate([hlo[...], hhi[...]], axis=1)
    u_ref[...] = (jnp.dot(h, wr[...], preferred_element_type=jnp.float32)
                  + b[...])


def _tc_right(hlo, hhi, wr, b):
    return pl.pallas_call(
        _tu_body,
        grid=(NBLK,),
        in_specs=[
            pl.BlockSpec((RB, H), lambda i: (i, 0)),
            pl.BlockSpec((RB, H), lambda i: (i, 0)),
            pl.BlockSpec((D, D), lambda i: (0, 0)),
            pl.BlockSpec((1, D), lambda i: (0, 0)),
        ],
        out_specs=pl.BlockSpec((RB, D), lambda i: (i, 0)),
        out_shape=jax.ShapeDtypeStruct((N, D), jnp.float32),
    )(hlo, hhi, wr, b)


def _t1_body(alo, ahi, u, cnta, cntb, wl,
             y_ref, stats_ref, stats_acc):
    i = pl.program_id(0)
    cnt = cnta[:, 0:1] + cntb[:, 0:1]
    inv = 1.0 / jnp.maximum(cnt, 1.0)
    agg = jnp.concatenate([alo[...], ahi[...]], axis=1) * inv
    y = (jnp.dot(agg, wl[...], preferred_element_type=jnp.float32)
         + u[...])
    y_ref[...] = y

    @pl.when(i == 0)
    def _():
        stats_acc[...] = jnp.zeros_like(stats_acc)

    s1 = jnp.sum(y, axis=0, keepdims=True)
    s2 = jnp.sum(y * y, axis=0, keepdims=True)
    stats_acc[0:1, :] += s1
    stats_acc[1:2, :] += s2

    @pl.when(i == NBLK - 1)
    def _():
        stats_ref[...] = stats_acc[...]


def _tc_matmul_stats(alo, ahi, u, cnta, cntb, wl):
    return pl.pallas_call(
        _t1_body,
        grid=(NBLK,),
        in_specs=[
            pl.BlockSpec((RB, H), lambda i: (i, 0)),
            pl.BlockSpec((RB, H), lambda i: (i, 0)),
            pl.BlockSpec((RB, D), lambda i: (i, 0)),
            pl.BlockSpec((RB, CW), lambda i: (i, 0)),
            pl.BlockSpec((RB, CW), lambda i: (i, 0)),
            pl.BlockSpec((D, D), lambda i: (0, 0)),
        ],
        out_specs=[
            pl.BlockSpec((RB, D), lambda i: (i, 0)),
            pl.BlockSpec((8, D), lambda i: (0, 0)),
        ],
        out_shape=[
            jax.ShapeDtypeStruct((N, D), jnp.float32),
            jax.ShapeDtypeStruct((8, D), jnp.float32),
        ],
        scratch_shapes=[pltpu.VMEM((8, D), jnp.float32)],
    )(alo, ahi, u, cnta, cntb, wl)


def _t2_body(y, stats, g, be, zlo_ref, zhi_ref):
    mu = stats[0:1, :] * (1.0 / N)
    var = stats[1:2, :] * (1.0 / N) - mu * mu
    scale = g[...] * lax.rsqrt(var + EPS)
    shift = be[...] - scale * mu
    z = jnp.maximum(y[...] * scale + shift, 0.0)
    zlo_ref[...] = z[:, :H]
    zhi_ref[...] = z[:, H:]


def _tc_norm_relu(y, stats, g, be):
    return pl.pallas_call(
        _t2_body,
        grid=(NBLK,),
        in_specs=[
            pl.BlockSpec((RB, D), lambda i: (i, 0)),
            pl.BlockSpec((8, D), lambda i: (0, 0)),
            pl.BlockSpec((1, D), lambda i: (0, 0)),
            pl.BlockSpec((1, D), lambda i: (0, 0)),
        ],
        out_specs=[
            pl.BlockSpec((RB, H), lambda i: (i, 0)),
            pl.BlockSpec((RB, H), lambda i: (i, 0)),
        ],
        out_shape=[
            jax.ShapeDtypeStruct((N, H), jnp.float32),
            jax.ShapeDtypeStruct((N, H), jnp.float32),
        ],
    )(y, stats, g, be)


def _t3_body(alo, ahi, u, cnta, cntb, wl, batch,
             out_ref, pool_acc, cg_acc):
    i = pl.program_id(0)
    cnt = cnta[:, 0:1] + cntb[:, 0:1]
    inv = 1.0 / jnp.maximum(cnt, 1.0)
    agg = jnp.concatenate([alo[...], ahi[...]], axis=1) * inv
    y = (jnp.dot(agg, wl[...], preferred_element_type=jnp.float32)
         + u[...])
    bb = batch[0, 0, :]
    oh = (bb[:, None] == lax.broadcasted_iota(jnp.int32, (RB, G), 1))
    oh = oh.astype(jnp.float32)

    @pl.when(i == 0)
    def _():
        pool_acc[...] = jnp.zeros_like(pool_acc)
        cg_acc[...] = jnp.zeros_like(cg_acc)

    pool_acc[...] += lax.dot_general(oh, y, (((0,), (0,)), ((), ())),
                                     preferred_element_type=jnp.float32)
    cg_acc[...] += lax.dot_general(oh, jnp.ones((RB, H), jnp.float32),
                                   (((0,), (0,)), ((), ())),
                                   preferred_element_type=jnp.float32)

    @pl.when(i == NBLK - 1)
    def _():
        out_ref[...] = pool_acc[...] / jnp.maximum(cg_acc[:, 0:1], 1.0)


def _tc_matmul_pool(alo, ahi, u, cnta, cntb, wl, batch3):
    return pl.pallas_call(
        _t3_body,
        grid=(NBLK,),
        in_specs=[
            pl.BlockSpec((RB, H), lambda i: (i, 0)),
            pl.BlockSpec((RB, H), lambda i: (i, 0)),
            pl.BlockSpec((RB, D), lambda i: (i, 0)),
            pl.BlockSpec((RB, CW), lambda i: (i, 0)),
            pl.BlockSpec((RB, CW), lambda i: (i, 0)),
            pl.BlockSpec((D, D), lambda i: (0, 0)),
            pl.BlockSpec((1, 1, RB), lambda i: (i, 0, 0)),
        ],
        out_specs=pl.BlockSpec((G, D), lambda i: (0, 0)),
        out_shape=jax.ShapeDtypeStruct((G, D), jnp.float32),
        scratch_shapes=[pltpu.VMEM((G, D), jnp.float32),
                        pltpu.VMEM((G, H), jnp.float32)],
    )(alo, ahi, u, cnta, cntb, wl, batch3)


def kernel(x, edge_index, batch, W1l, W1r, b1, g1, be1,
           W2l, W2r, b2, g2, be2, W3l, W3r, b3):
    xlo = x[:, :H]
    xhi = x[:, H:]
    src2 = edge_index[0].reshape(E // CH, CH)
    dst2 = edge_index[1].reshape(E // CH, CH)
    dstc = edge_index[1].reshape(E // CCH, CCH)
    zeros128 = jnp.zeros((N, H), jnp.float32)
    zeros16 = jnp.zeros((N, CW), jnp.float32)
    ones16 = jnp.ones((CCH, CW), jnp.float32)
    batch3 = batch.reshape(NBLK, 1, RB)
    b1r = b1.reshape(1, D)
    g1r = g1.reshape(1, D)
    be1r = be1.reshape(1, D)
    b2r = b2.reshape(1, D)
    g2r = g2.reshape(1, D)
    be2r = be2.reshape(1, D)
    b3r = b3.reshape(1, D)

    _sc_aggregate, _sc_counts = _sc_kernels()

    cnta, cntb = _sc_counts(dstc, ones16, zeros16)

    a1lo, a1hi = _sc_aggregate(xlo, xhi, src2, dst2, zeros128)
    u1 = _tc_right(xlo, xhi, W1r, b1r)
    y1, st1 = _tc_matmul_stats(a1lo, a1hi, u1, cnta, cntb, W1l)
    h1lo, h1hi = _tc_norm_relu(y1, st1, g1r, be1r)

    a2lo, a2hi = _sc_aggregate(h1lo, h1hi, src2, dst2, zeros128)
    u2 = _tc_right(h1lo, h1hi, W2r, b2r)
    y2, st2 = _tc_matmul_stats(a2lo, a2hi, u2, cnta, cntb, W2l)
    h2lo, h2hi = _tc_norm_relu(y2, st2, g2r, be2r)

    a3lo, a3hi = _sc_aggregate(h2lo, h2hi, src2, dst2, zeros128)
    u3 = _tc_right(h2lo, h2hi, W3r, b3r)
    return _tc_matmul_pool(a3lo, a3hi, u3, cnta, cntb, W3l, batch3)


# speedup vs baseline: 1.0335x; 1.0335x over previous
import functools

import jax
import jax.numpy as jnp
from jax import lax
from jax.experimental import pallas as pl
from jax.experimental.pallas import tpu as pltpu
from jax.experimental.pallas import tpu_sc as plsc

N = 10000
E = 160000
D = 256
H = 128
G = 64
EPS = 1e-5

NC = 2
NS = 16

CH = 125
NCHUNK = (E // NS) // CH
HCH = 40
WB = 640
WB_LAST = N - WB * (NS - 1)

CCH = 125
CROWS = (E // (NC * NS)) // CCH
CW = 128

@functools.cache
def _sc_kernels():
    mesh = plsc.VectorSubcoreMesh(core_axis_name="c", subcore_axis_name="s",
                                  num_cores=NC, num_subcores=NS)

    def shard_copy(src_ref, dst_ref, s, **kw):
        pl.when(s < NS - 1)(lambda: pltpu.sync_copy(
            src_ref.at[pl.ds(s * WB, WB)], dst_ref.at[pl.ds(s * WB, WB)], **kw))
        pl.when(s == NS - 1)(lambda: pltpu.sync_copy(
            src_ref.at[pl.ds((NS - 1) * WB, WB_LAST)],
            dst_ref.at[pl.ds((NS - 1) * WB, WB_LAST)], **kw))

    @functools.partial(
        pl.kernel,
        out_type=[jax.ShapeDtypeStruct((N, H), jnp.float32),
                  jax.ShapeDtypeStruct((N, H), jnp.float32)],
        mesh=mesh,
        scratch_types=[
            pltpu.VMEM((HCH, CH), jnp.int32),
            pltpu.VMEM((HCH, CH), jnp.int32),
            pltpu.VMEM((CH, H), jnp.float32),
            pltpu.VMEM((CH, H), jnp.float32),
            pltpu.VMEM_SHARED((N, H), jnp.float32),
            pltpu.SemaphoreType.DMA,
            pltpu.SemaphoreType.DMA,
            pltpu.SemaphoreType.DMA,
        ],
    )
    def sc_aggregate(xlo_hbm, xhi_hbm, src_hbm, dst_hbm, zeros_hbm,
                     alo_hbm, ahi_hbm,
                     src_v, dst_v, rows_v0, rows_v1, acc_sh,
                     sem0, sem1, semz):
        c = lax.axis_index("c")
        s = lax.axis_index("s")
        def _zero_start_main():
            pltpu.async_copy(zeros_hbm.at[pl.ds(s * WB, WB)],
                             acc_sh.at[pl.ds(s * WB, WB)], semz)

        def _zero_start_last():
            pltpu.async_copy(zeros_hbm.at[pl.ds((NS - 1) * WB, WB_LAST)],
                             acc_sh.at[pl.ds((NS - 1) * WB, WB_LAST)], semz)

        pl.when(s < NS - 1)(_zero_start_main)
        pl.when(s == NS - 1)(_zero_start_last)

        def run(x_hbm):
            def g_start(j, buf, sem):
                pltpu.async_copy(x_hbm.at[src_v.at[j]], buf, sem)

            def g_wait(buf, sem):
                pltpu.make_async_copy(x_hbm.at[src_v.at[0]], buf, sem).wait()

            def stage_and_prime(hbase):
                pltpu.sync_copy(src_hbm.at[pl.ds(hbase, HCH)], src_v)
                pltpu.sync_copy(dst_hbm.at[pl.ds(hbase, HCH)], dst_v)
                g_start(0, rows_v0, sem0)
                g_start(1, rows_v1, sem1)

            def inner():
                def step(jj, carry):
                    j0 = jj * 2
                    j1 = j0 + 1
                    g_wait(rows_v0, sem0)
                    pltpu.sync_copy(rows_v0, acc_sh.at[dst_v.at[j0]],
                                    add=True)
                    pl.when(j0 + 2 < HCH)(
                        lambda: g_start(j0 + 2, rows_v0, sem0))
                    g_wait(rows_v1, sem1)
                    pltpu.sync_copy(rows_v1, acc_sh.at[dst_v.at[j1]],
                                    add=True)
                    pl.when(j1 + 2 < HCH)(
                        lambda: g_start(j1 + 2, rows_v1, sem1))
                    return carry
                lax.fori_loop(0, HCH // 2, step, 0)

            stage_and_prime(s * NCHUNK)

            def _zero_wait_main():
                pltpu.make_async_copy(
                    zeros_hbm.at[pl.ds(s * WB, WB)],
                    acc_sh.at[pl.ds(s * WB, WB)], semz).wait()

            def _zero_wait_last():
                pltpu.make_async_copy(
                    zeros_hbm.at[pl.ds((NS - 1) * WB, WB_LAST)],
                    acc_sh.at[pl.ds((NS - 1) * WB, WB_LAST)], semz).wait()

            pl.when(s < NS - 1)(_zero_wait_main)
            pl.when(s == NS - 1)(_zero_wait_last)
            plsc.subcore_barrier()
            inner()
            for hh in range(1, NCHUNK // HCH):
                stage_and_prime(s * NCHUNK + hh * HCH)
                inner()

        pl.when(c == 0)(lambda: run(xlo_hbm))
        pl.when(c == 1)(lambda: run(xhi_hbm))
        plsc.subcore_barrier()

        pl.when(c == 0)(lambda: shard_copy(acc_sh, alo_hbm, s))
        pl.when(c == 1)(lambda: shard_copy(acc_sh, ahi_hbm, s))

    @functools.partial(
        pl.kernel,
        out_type=[jax.ShapeDtypeStruct((N, CW), jnp.float32),
                  jax.ShapeDtypeStruct((N, CW), jnp.float32)],
        mesh=mesh,
        scratch_types=[
            pltpu.VMEM((CROWS, CCH), jnp.int32),
            pltpu.VMEM((CCH, CW), jnp.float32),
            pltpu.VMEM_SHARED((N, CW), jnp.float32),
        ],
    )
    def sc_counts(dst_hbm, ones_hbm, zeros_hbm, cnta_hbm, cntb_hbm,
                  dst_v, ones_v, cnt_sh):
        c = lax.axis_index("c")
        s = lax.axis_index("s")
        w = s * NC + c
        shard_copy(zeros_hbm, cnt_sh, s)
        pltpu.sync_copy(ones_hbm, ones_v)
        pltpu.sync_copy(dst_hbm.at[pl.ds(w * CROWS, CROWS)], dst_v)
        plsc.subcore_barrier()

        def step(j, carry):
            pltpu.sync_copy(ones_v, cnt_sh.at[dst_v.at[j]], add=True)
            return carry
        lax.fori_loop(0, CROWS, step, 0)
        plsc.subcore_barrier()

        pl.when(c == 0)(lambda: shard_copy(cnt_sh, cnta_hbm, s))
        pl.when(c == 1)(lambda: shard_copy(cnt_sh, cntb_hbm, s))

    return sc_aggregate, sc_counts


RB = 1000
NBLK = N // RB


def _t1_body(alo, ahi, hlo, hhi, cnta, cntb, wl, wr, b,
             y_ref, stats_ref, stats_acc):
    i = pl.program_id(0)
    cnt = cnta[:, 0:1] + cntb[:, 0:1]
    inv = 1.0 / jnp.maximum(cnt, 1.0)
    agg = jnp.concatenate([alo[...], ahi[...]], axis=1) * inv
    h = jnp.concatenate([hlo[...], hhi[...]], axis=1)
    y = (jnp.dot(agg, wl[...], preferred_element_type=jnp.float32)
         + jnp.dot(h, wr[...], preferred_element_type=jnp.float32)
         + b[...])
    y_ref[...] = y

    @pl.when(i == 0)
    def _():
        stats_acc[...] = jnp.zeros_like(stats_acc)

    s1 = jnp.sum(y, axis=0, keepdims=True)
    s2 = jnp.sum(y * y, axis=0, keepdims=True)
    stats_acc[0:1, :] += s1
    stats_acc[1:2, :] += s2

    @pl.when(i == NBLK - 1)
    def _():
        stats_ref[...] = stats_acc[...]


def _tc_matmul_stats(alo, ahi, hlo, hhi, cnta, cntb, wl, wr, b):
    return pl.pallas_call(
        _t1_body,
        grid=(NBLK,),
        in_specs=[
            pl.BlockSpec((RB, H), lambda i: (i, 0)),
            pl.BlockSpec((RB, H), lambda i: (i, 0)),
            pl.BlockSpec((RB, H), lambda i: (i, 0)),
            pl.BlockSpec((RB, H), lambda i: (i, 0)),
            pl.BlockSpec((RB, CW), lambda i: (i, 0)),
            pl.BlockSpec((RB, CW), lambda i: (i, 0)),
            pl.BlockSpec((D, D), lambda i: (0, 0)),
            pl.BlockSpec((D, D), lambda i: (0, 0)),
            pl.BlockSpec((1, D), lambda i: (0, 0)),
        ],
        out_specs=[
            pl.BlockSpec((RB, D), lambda i: (i, 0)),
            pl.BlockSpec((8, D), lambda i: (0, 0)),
        ],
        out_shape=[
            jax.ShapeDtypeStruct((N, D), jnp.float32),
            jax.ShapeDtypeStruct((8, D), jnp.float32),
        ],
        scratch_shapes=[pltpu.VMEM((8, D), jnp.float32)],
    )(alo, ahi, hlo, hhi, cnta, cntb, wl, wr, b)


def _t2_body(y, stats, g, be, zlo_ref, zhi_ref):
    mu = stats[0:1, :] * (1.0 / N)
    var = stats[1:2, :] * (1.0 / N) - mu * mu
    scale = g[...] * lax.rsqrt(var + EPS)
    shift = be[...] - scale * mu
    z = jnp.maximum(y[...] * scale + shift, 0.0)
    zlo_ref[...] = z[:, :H]
    zhi_ref[...] = z[:, H:]


def _tc_norm_relu(y, stats, g, be):
    return pl.pallas_call(
        _t2_body,
        grid=(NBLK,),
        in_specs=[
            pl.BlockSpec((RB, D), lambda i: (i, 0)),
            pl.BlockSpec((8, D), lambda i: (0, 0)),
            pl.BlockSpec((1, D), lambda i: (0, 0)),
            pl.BlockSpec((1, D), lambda i: (0, 0)),
        ],
        out_specs=[
            pl.BlockSpec((RB, H), lambda i: (i, 0)),
            pl.BlockSpec((RB, H), lambda i: (i, 0)),
        ],
        out_shape=[
            jax.ShapeDtypeStruct((N, H), jnp.float32),
            jax.ShapeDtypeStruct((N, H), jnp.float32),
        ],
    )(y, stats, g, be)


def _t3_body(alo, ahi, hlo, hhi, cnta, cntb, wl, wr, b, batch,
             out_ref, pool_acc, cg_acc):
    i = pl.program_id(0)
    cnt = cnta[:, 0:1] + cntb[:, 0:1]
    inv = 1.0 / jnp.maximum(cnt, 1.0)
    agg = jnp.concatenate([alo[...], ahi[...]], axis=1) * inv
    h = jnp.concatenate([hlo[...], hhi[...]], axis=1)
    y = (jnp.dot(agg, wl[...], preferred_element_type=jnp.float32)
         + jnp.dot(h, wr[...], preferred_element_type=jnp.float32)
         + b[...])
    bb = batch[0, 0, :]
    oh = (bb[:, None] == lax.broadcasted_iota(jnp.int32, (RB, G), 1))
    oh = oh.astype(jnp.float32)

    @pl.when(i == 0)
    def _():
        pool_acc[...] = jnp.zeros_like(pool_acc)
        cg_acc[...] = jnp.zeros_like(cg_acc)

    pool_acc[...] += lax.dot_general(oh, y, (((0,), (0,)), ((), ())),
                                     preferred_element_type=jnp.float32)
    cg_acc[...] += lax.dot_general(oh, jnp.ones((RB, H), jnp.float32),
                                   (((0,), (0,)), ((), ())),
                                   preferred_element_type=jnp.float32)

    @pl.when(i == NBLK - 1)
    def _():
        out_ref[...] = pool_acc[...] / jnp.maximum(cg_acc[:, 0:1], 1.0)


def _tc_matmul_pool(alo, ahi, hlo, hhi, cnta, cntb, wl, wr, b, batch3):
    return pl.pallas_call(
        _t3_body,
        grid=(NBLK,),
        in_specs=[
            pl.BlockSpec((RB, H), lambda i: (i, 0)),
            pl.BlockSpec((RB, H), lambda i: (i, 0)),
            pl.BlockSpec((RB, H), lambda i: (i, 0)),
            pl.BlockSpec((RB, H), lambda i: (i, 0)),
            pl.BlockSpec((RB, CW), lambda i: (i, 0)),
            pl.BlockSpec((RB, CW), lambda i: (i, 0)),
            pl.BlockSpec((D, D), lambda i: (0, 0)),
            pl.BlockSpec((D, D), lambda i: (0, 0)),
            pl.BlockSpec((1, D), lambda i: (0, 0)),
            pl.BlockSpec((1, 1, RB), lambda i: (i, 0, 0)),
        ],
        out_specs=pl.BlockSpec((G, D), lambda i: (0, 0)),
        out_shape=jax.ShapeDtypeStruct((G, D), jnp.float32),
        scratch_shapes=[pltpu.VMEM((G, D), jnp.float32),
                        pltpu.VMEM((G, H), jnp.float32)],
    )(alo, ahi, hlo, hhi, cnta, cntb, wl, wr, b, batch3)


def kernel(x, edge_index, batch, W1l, W1r, b1, g1, be1,
           W2l, W2r, b2, g2, be2, W3l, W3r, b3):
    xlo = x[:, :H]
    xhi = x[:, H:]
    src2 = edge_index[0].reshape(E // CH, CH)
    dst2 = edge_index[1].reshape(E // CH, CH)
    dstc = edge_index[1].reshape(E // CCH, CCH)
    zeros128 = jnp.zeros((N, H), jnp.float32)
    zeros16 = jnp.zeros((N, CW), jnp.float32)
    ones16 = jnp.ones((CCH, CW), jnp.float32)
    batch3 = batch.reshape(NBLK, 1, RB)
    b1r = b1.reshape(1, D)
    g1r = g1.reshape(1, D)
    be1r = be1.reshape(1, D)
    b2r = b2.reshape(1, D)
    g2r = g2.reshape(1, D)
    be2r = be2.reshape(1, D)
    b3r = b3.reshape(1, D)

    _sc_aggregate, _sc_counts = _sc_kernels()

    cnta, cntb = _sc_counts(dstc, ones16, zeros16)

    a1lo, a1hi = _sc_aggregate(xlo, xhi, src2, dst2, zeros128)
    y1, st1 = _tc_matmul_stats(a1lo, a1hi, xlo, xhi, cnta, cntb,
                               W1l, W1r, b1r)
    h1lo, h1hi = _tc_norm_relu(y1, st1, g1r, be1r)

    a2lo, a2hi = _sc_aggregate(h1lo, h1hi, src2, dst2, zeros128)
    y2, st2 = _tc_matmul_stats(a2lo, a2hi, h1lo, h1hi, cnta, cntb,
                               W2l, W2r, b2r)
    h2lo, h2hi = _tc_norm_relu(y2, st2, g2r, be2r)

    a3lo, a3hi = _sc_aggregate(h2lo, h2hi, src2, dst2, zeros128)
    return _tc_matmul_pool(a3lo, a3hi, h2lo, h2hi, cnta, cntb,
                           W3l, W3r, b3r, batch3)


# speedup vs baseline: 1.1410x; 1.1040x over previous
import functools

import jax
import jax.numpy as jnp
from jax import lax
from jax.experimental import pallas as pl
from jax.experimental.pallas import tpu as pltpu
from jax.experimental.pallas import tpu_sc as plsc

N = 10000
E = 160000
D = 256
H = 128
G = 64
EPS = 1e-5

NC = 2
NS = 16

CH = 125
NCHUNK = (E // NS) // CH
HCH = 40
WB = 640
WB_LAST = N - WB * (NS - 1)

HR = 80
EPT = E // NS

@functools.cache
def _sc_kernels():
    mesh = plsc.VectorSubcoreMesh(core_axis_name="c", subcore_axis_name="s",
                                  num_cores=NC, num_subcores=NS)

    def shard_copy(src_ref, dst_ref, s, **kw):
        pl.when(s < NS - 1)(lambda: pltpu.sync_copy(
            src_ref.at[pl.ds(s * WB, WB)], dst_ref.at[pl.ds(s * WB, WB)], **kw))
        pl.when(s == NS - 1)(lambda: pltpu.sync_copy(
            src_ref.at[pl.ds((NS - 1) * WB, WB_LAST)],
            dst_ref.at[pl.ds((NS - 1) * WB, WB_LAST)], **kw))

    @functools.partial(
        pl.kernel,
        out_type=[jax.ShapeDtypeStruct((N, H), jnp.float32),
                  jax.ShapeDtypeStruct((N, H), jnp.float32)],
        mesh=mesh,
        scratch_types=[
            pltpu.VMEM((HCH, CH), jnp.int32),
            pltpu.VMEM((HCH, CH), jnp.int32),
            pltpu.VMEM((CH, H), jnp.float32),
            pltpu.VMEM((CH, H), jnp.float32),
            pltpu.VMEM_SHARED((N, H), jnp.float32),
            pltpu.SemaphoreType.DMA,
            pltpu.SemaphoreType.DMA,
            pltpu.SemaphoreType.DMA,
        ],
    )
    def sc_aggregate(xlo_hbm, xhi_hbm, src_hbm, dst_hbm, zeros_hbm,
                     alo_hbm, ahi_hbm,
                     src_v, dst_v, rows_v0, rows_v1, acc_sh,
                     sem0, sem1, semz):
        c = lax.axis_index("c")
        s = lax.axis_index("s")
        def _zero_start_main():
            pltpu.async_copy(zeros_hbm.at[pl.ds(s * WB, WB)],
                             acc_sh.at[pl.ds(s * WB, WB)], semz)

        def _zero_start_last():
            pltpu.async_copy(zeros_hbm.at[pl.ds((NS - 1) * WB, WB_LAST)],
                             acc_sh.at[pl.ds((NS - 1) * WB, WB_LAST)], semz)

        pl.when(s < NS - 1)(_zero_start_main)
        pl.when(s == NS - 1)(_zero_start_last)

        def run(x_hbm):
            def g_start(j, buf, sem):
                pltpu.async_copy(x_hbm.at[src_v.at[j]], buf, sem)

            def g_wait(buf, sem):
                pltpu.make_async_copy(x_hbm.at[src_v.at[0]], buf, sem).wait()

            def stage_and_prime(hbase):
                pltpu.sync_copy(src_hbm.at[pl.ds(hbase, HCH)], src_v)
                pltpu.sync_copy(dst_hbm.at[pl.ds(hbase, HCH)], dst_v)
                g_start(0, rows_v0, sem0)
                g_start(1, rows_v1, sem1)

            def inner():
                def step(jj, carry):
                    j0 = jj * 2
                    j1 = j0 + 1
                    g_wait(rows_v0, sem0)
                    pltpu.sync_copy(rows_v0, acc_sh.at[dst_v.at[j0]],
                                    add=True)
                    pl.when(j0 + 2 < HCH)(
                        lambda: g_start(j0 + 2, rows_v0, sem0))
                    g_wait(rows_v1, sem1)
                    pltpu.sync_copy(rows_v1, acc_sh.at[dst_v.at[j1]],
                                    add=True)
                    pl.when(j1 + 2 < HCH)(
                        lambda: g_start(j1 + 2, rows_v1, sem1))
                    return carry
                lax.fori_loop(0, HCH // 2, step, 0)

            stage_and_prime(s * NCHUNK)

            def _zero_wait_main():
                pltpu.make_async_copy(
                    zeros_hbm.at[pl.ds(s * WB, WB)],
                    acc_sh.at[pl.ds(s * WB, WB)], semz).wait()

            def _zero_wait_last():
                pltpu.make_async_copy(
                    zeros_hbm.at[pl.ds((NS - 1) * WB, WB_LAST)],
                    acc_sh.at[pl.ds((NS - 1) * WB, WB_LAST)], semz).wait()

            pl.when(s < NS - 1)(_zero_wait_main)
            pl.when(s == NS - 1)(_zero_wait_last)
            plsc.subcore_barrier()
            inner()
            for hh in range(1, NCHUNK // HCH):
                stage_and_prime(s * NCHUNK + hh * HCH)
                inner()

        pl.when(c == 0)(lambda: run(xlo_hbm))
        pl.when(c == 1)(lambda: run(xhi_hbm))
        plsc.subcore_barrier()

        pl.when(c == 0)(lambda: shard_copy(acc_sh, alo_hbm, s))
        pl.when(c == 1)(lambda: shard_copy(acc_sh, ahi_hbm, s))

    @functools.partial(
        pl.kernel,
        out_type=jax.ShapeDtypeStruct((HR, H), jnp.float32),
        mesh=mesh,
        compiler_params=pltpu.CompilerParams(needs_layout_passes=False),
        scratch_types=[
            pltpu.VMEM((EPT,), jnp.int32),
            pltpu.VMEM((HR, H), jnp.float32),
            pltpu.VMEM((HR,), jnp.int32),
            pltpu.VMEM((8, H), jnp.float32),
            pltpu.VMEM_SHARED((HR, H), jnp.float32),
        ],
    )
    def sc_counts(dst_hbm, zeros_hbm, inv_hbm,
                  dst_v, cnt2d, idx80, bufr, acc_sh):
        c = lax.axis_index("c")
        s = lax.axis_index("s")
        pl.when(s < HR // 8)(lambda: pltpu.sync_copy(
            zeros_hbm.at[pl.ds(s * 8, 8)], acc_sh.at[pl.ds(s * 8, 8)]))
        pltpu.sync_copy(dst_hbm.at[pl.ds(s * EPT, EPT)], dst_v)
        for k in range(HR // 16):
            idx80[pl.ds(k * 16, 16)] = lax.iota(jnp.int32, 16) + (16 * k)
        zv = jnp.zeros((16,), jnp.float32)

        def zstep(i, carry):
            r = i // 8
            k = i - r * 8
            cnt2d[r, pl.ds(k * 16, 16)] = zv
            return carry
        lax.fori_loop(0, HR * 8, zstep, 0)

        ones = jnp.ones((16,), jnp.float32)

        def estep(i, carry):
            idx = dst_v[pl.ds(i * 16, 16)]
            row = lax.shift_right_logical(idx, 7)
            col = jnp.bitwise_and(idx, 127)
            plsc.addupdate_scatter(cnt2d, [row, col], ones)
            return carry
        lax.fori_loop(0, EPT // 16, estep, 0)

        plsc.subcore_barrier()
        pltpu.sync_copy(cnt2d, acc_sh.at[idx80], add=True)
        plsc.subcore_barrier()

        def _out():
            g = c * (HR // 16) + s
            pltpu.sync_copy(acc_sh.at[pl.ds(g * 8, 8)], bufr)
            for r in range(8):
                for k in range(8):
                    v = bufr[r, pl.ds(k * 16, 16)]
                    bufr[r, pl.ds(k * 16, 16)] = 1.0 / jnp.maximum(v, 1.0)
            pltpu.sync_copy(bufr, inv_hbm.at[pl.ds(g * 8, 8)])

        pl.when(s < HR // 16)(_out)

    return sc_aggregate, sc_counts


RB = 1000
NBLK = N // RB


def _t1_body(alo, ahi, hlo, hhi, inv, wl, wr, b,
             y_ref, stats_ref, stats_acc):
    i = pl.program_id(0)
    agg = jnp.concatenate([alo[...], ahi[...]], axis=1) * inv[...]
    h = jnp.concatenate([hlo[...], hhi[...]], axis=1)
    y = (jnp.dot(agg, wl[...], preferred_element_type=jnp.float32)
         + jnp.dot(h, wr[...], preferred_element_type=jnp.float32)
         + b[...])
    y_ref[...] = y

    @pl.when(i == 0)
    def _():
        stats_acc[...] = jnp.zeros_like(stats_acc)

    s1 = jnp.sum(y, axis=0, keepdims=True)
    s2 = jnp.sum(y * y, axis=0, keepdims=True)
    stats_acc[0:1, :] += s1
    stats_acc[1:2, :] += s2

    @pl.when(i == NBLK - 1)
    def _():
        stats_ref[...] = stats_acc[...]


def _tc_matmul_stats(alo, ahi, hlo, hhi, inv2d, wl, wr, b):
    return pl.pallas_call(
        _t1_body,
        grid=(NBLK,),
        in_specs=[
            pl.BlockSpec((RB, H), lambda i: (i, 0)),
            pl.BlockSpec((RB, H), lambda i: (i, 0)),
            pl.BlockSpec((RB, H), lambda i: (i, 0)),
            pl.BlockSpec((RB, H), lambda i: (i, 0)),
            pl.BlockSpec((RB, 1), lambda i: (i, 0)),
            pl.BlockSpec((D, D), lambda i: (0, 0)),
            pl.BlockSpec((D, D), lambda i: (0, 0)),
            pl.BlockSpec((1, D), lambda i: (0, 0)),
        ],
        out_specs=[
            pl.BlockSpec((RB, D), lambda i: (i, 0)),
            pl.BlockSpec((8, D), lambda i: (0, 0)),
        ],
        out_shape=[
            jax.ShapeDtypeStruct((N, D), jnp.float32),
            jax.ShapeDtypeStruct((8, D), jnp.float32),
        ],
        scratch_shapes=[pltpu.VMEM((8, D), jnp.float32)],
    )(alo, ahi, hlo, hhi, inv2d, wl, wr, b)


def _t2_body(y, stats, g, be, zlo_ref, zhi_ref):
    mu = stats[0:1, :] * (1.0 / N)
    var = stats[1:2, :] * (1.0 / N) - mu * mu
    scale = g[...] * lax.rsqrt(var + EPS)
    shift = be[...] - scale * mu
    z = jnp.maximum(y[...] * scale + shift, 0.0)
    zlo_ref[...] = z[:, :H]
    zhi_ref[...] = z[:, H:]


def _tc_norm_relu(y, stats, g, be):
    return pl.pallas_call(
        _t2_body,
        grid=(NBLK,),
        in_specs=[
            pl.BlockSpec((RB, D), lambda i: (i, 0)),
            pl.BlockSpec((8, D), lambda i: (0, 0)),
            pl.BlockSpec((1, D), lambda i: (0, 0)),
            pl.BlockSpec((1, D), lambda i: (0, 0)),
        ],
        out_specs=[
            pl.BlockSpec((RB, H), lambda i: (i, 0)),
            pl.BlockSpec((RB, H), lambda i: (i, 0)),
        ],
        out_shape=[
            jax.ShapeDtypeStruct((N, H), jnp.float32),
            jax.ShapeDtypeStruct((N, H), jnp.float32),
        ],
    )(y, stats, g, be)


def _t3_body(alo, ahi, hlo, hhi, inv, wl, wr, b, batch,
             out_ref, pool_acc, cg_acc):
    i = pl.program_id(0)
    agg = jnp.concatenate([alo[...], ahi[...]], axis=1) * inv[...]
    h = jnp.concatenate([hlo[...], hhi[...]], axis=1)
    y = (jnp.dot(agg, wl[...], preferred_element_type=jnp.float32)
         + jnp.dot(h, wr[...], preferred_element_type=jnp.float32)
         + b[...])
    bb = batch[0, 0, :]
    oh = (bb[:, None] == lax.broadcasted_iota(jnp.int32, (RB, G), 1))
    oh = oh.astype(jnp.float32)

    @pl.when(i == 0)
    def _():
        pool_acc[...] = jnp.zeros_like(pool_acc)
        cg_acc[...] = jnp.zeros_like(cg_acc)

    pool_acc[...] += lax.dot_general(oh, y, (((0,), (0,)), ((), ())),
                                     preferred_element_type=jnp.float32)
    cg_acc[...] += lax.dot_general(oh, jnp.ones((RB, H), jnp.float32),
                                   (((0,), (0,)), ((), ())),
                                   preferred_element_type=jnp.float32)

    @pl.when(i == NBLK - 1)
    def _():
        out_ref[...] = pool_acc[...] / jnp.maximum(cg_acc[:, 0:1], 1.0)


def _tc_matmul_pool(alo, ahi, hlo, hhi, inv2d, wl, wr, b, batch3):
    return pl.pallas_call(
        _t3_body,
        grid=(NBLK,),
        in_specs=[
            pl.BlockSpec((RB, H), lambda i: (i, 0)),
            pl.BlockSpec((RB, H), lambda i: (i, 0)),
            pl.BlockSpec((RB, H), lambda i: (i, 0)),
            pl.BlockSpec((RB, H), lambda i: (i, 0)),
            pl.BlockSpec((RB, 1), lambda i: (i, 0)),
            pl.BlockSpec((D, D), lambda i: (0, 0)),
            pl.BlockSpec((D, D), lambda i: (0, 0)),
            pl.BlockSpec((1, D), lambda i: (0, 0)),
            pl.BlockSpec((1, 1, RB), lambda i: (i, 0, 0)),
        ],
        out_specs=pl.BlockSpec((G, D), lambda i: (0, 0)),
        out_shape=jax.ShapeDtypeStruct((G, D), jnp.float32),
        scratch_shapes=[pltpu.VMEM((G, D), jnp.float32),
                        pltpu.VMEM((G, H), jnp.float32)],
    )(alo, ahi, hlo, hhi, inv2d, wl, wr, b, batch3)


def kernel(x, edge_index, batch, W1l, W1r, b1, g1, be1,
           W2l, W2r, b2, g2, be2, W3l, W3r, b3):
    xlo = x[:, :H]
    xhi = x[:, H:]
    src2 = edge_index[0].reshape(E // CH, CH)
    dst2 = edge_index[1].reshape(E // CH, CH)
    dst1 = edge_index[1]
    zeros128 = jnp.zeros((N, H), jnp.float32)
    batch3 = batch.reshape(NBLK, 1, RB)
    b1r = b1.reshape(1, D)
    g1r = g1.reshape(1, D)
    be1r = be1.reshape(1, D)
    b2r = b2.reshape(1, D)
    g2r = g2.reshape(1, D)
    be2r = be2.reshape(1, D)
    b3r = b3.reshape(1, D)

    _sc_aggregate, _sc_counts = _sc_kernels()

    inv80 = _sc_counts(dst1, zeros128)
    inv2d = inv80.reshape(HR * H)[:N].reshape(N, 1)

    a1lo, a1hi = _sc_aggregate(xlo, xhi, src2, dst2, zeros128)
    y1, st1 = _tc_matmul_stats(a1lo, a1hi, xlo, xhi, inv2d,
                               W1l, W1r, b1r)
    h1lo, h1hi = _tc_norm_relu(y1, st1, g1r, be1r)

    a2lo, a2hi = _sc_aggregate(h1lo, h1hi, src2, dst2, zeros128)
    y2, st2 = _tc_matmul_stats(a2lo, a2hi, h1lo, h1hi, inv2d,
                               W2l, W2r, b2r)
    h2lo, h2hi = _tc_norm_relu(y2, st2, g2r, be2r)

    a3lo, a3hi = _sc_aggregate(h2lo, h2hi, src2, dst2, zeros128)
    return _tc_matmul_pool(a3lo, a3hi, h2lo, h2hi, inv2d,
                           W3l, W3r, b3r, batch3)


# speedup vs baseline: 1.1652x; 1.0213x over previous
import functools

import jax
import jax.numpy as jnp
from jax import lax
from jax.experimental import pallas as pl
from jax.experimental.pallas import tpu as pltpu
from jax.experimental.pallas import tpu_sc as plsc

N = 10000
E = 160000
D = 256
H = 128
G = 64
EPS = 1e-5

NC = 2
NS = 16

CH = 125
NCHUNK = (E // NS) // CH
HCH = 40
WB = 640
WB_LAST = N - WB * (NS - 1)

HR = 80
EPT = E // NS

@functools.cache
def _sc_kernels():
    mesh = plsc.VectorSubcoreMesh(core_axis_name="c", subcore_axis_name="s",
                                  num_cores=NC, num_subcores=NS)

    def shard_copy(src_ref, dst_ref, s, **kw):
        pl.when(s < NS - 1)(lambda: pltpu.sync_copy(
            src_ref.at[pl.ds(s * WB, WB)], dst_ref.at[pl.ds(s * WB, WB)], **kw))
        pl.when(s == NS - 1)(lambda: pltpu.sync_copy(
            src_ref.at[pl.ds((NS - 1) * WB, WB_LAST)],
            dst_ref.at[pl.ds((NS - 1) * WB, WB_LAST)], **kw))

    @functools.partial(
        pl.kernel,
        out_type=[jax.ShapeDtypeStruct((N, H), jnp.float32),
                  jax.ShapeDtypeStruct((N, H), jnp.float32)],
        mesh=mesh,
        scratch_types=[
            pltpu.VMEM((HCH, CH), jnp.int32),
            pltpu.VMEM((HCH, CH), jnp.int32),
            pltpu.VMEM((CH, H), jnp.float32),
            pltpu.VMEM((CH, H), jnp.float32),
            pltpu.VMEM_SHARED((N, H), jnp.float32),
            pltpu.SemaphoreType.DMA,
            pltpu.SemaphoreType.DMA,
            pltpu.SemaphoreType.DMA,
        ],
    )
    def sc_aggregate(xlo_hbm, xhi_hbm, src_hbm, dst_hbm, zeros_hbm,
                     alo_hbm, ahi_hbm,
                     src_v, dst_v, rows_v0, rows_v1, acc_sh,
                     sem0, sem1, semz):
        c = lax.axis_index("c")
        s = lax.axis_index("s")
        def _zero_start_main():
            pltpu.async_copy(zeros_hbm.at[pl.ds(s * WB, WB)],
                             acc_sh.at[pl.ds(s * WB, WB)], semz)

        def _zero_start_last():
            pltpu.async_copy(zeros_hbm.at[pl.ds((NS - 1) * WB, WB_LAST)],
                             acc_sh.at[pl.ds((NS - 1) * WB, WB_LAST)], semz)

        pl.when(s < NS - 1)(_zero_start_main)
        pl.when(s == NS - 1)(_zero_start_last)

        def run(x_hbm):
            def g_start(j, buf, sem):
                pltpu.async_copy(x_hbm.at[src_v.at[j]], buf, sem)

            def g_wait(buf, sem):
                pltpu.make_async_copy(x_hbm.at[src_v.at[0]], buf, sem).wait()

            def stage_and_prime(hbase):
                pltpu.sync_copy(src_hbm.at[pl.ds(hbase, HCH)], src_v)
                pltpu.sync_copy(dst_hbm.at[pl.ds(hbase, HCH)], dst_v)
                g_start(0, rows_v0, sem0)
                g_start(1, rows_v1, sem1)

            def inner():
                def step(jj, carry):
                    j0 = jj * 2
                    j1 = j0 + 1
                    g_wait(rows_v0, sem0)
                    pltpu.sync_copy(rows_v0, acc_sh.at[dst_v.at[j0]],
                                    add=True)
                    pl.when(j0 + 2 < HCH)(
                        lambda: g_start(j0 + 2, rows_v0, sem0))
                    g_wait(rows_v1, sem1)
                    pltpu.sync_copy(rows_v1, acc_sh.at[dst_v.at[j1]],
                                    add=True)
                    pl.when(j1 + 2 < HCH)(
                        lambda: g_start(j1 + 2, rows_v1, sem1))
                    return carry
                lax.fori_loop(0, HCH // 2, step, 0)

            stage_and_prime(s * NCHUNK)

            def _zero_wait_main():
                pltpu.make_async_copy(
                    zeros_hbm.at[pl.ds(s * WB, WB)],
                    acc_sh.at[pl.ds(s * WB, WB)], semz).wait()

            def _zero_wait_last():
                pltpu.make_async_copy(
                    zeros_hbm.at[pl.ds((NS - 1) * WB, WB_LAST)],
                    acc_sh.at[pl.ds((NS - 1) * WB, WB_LAST)], semz).wait()

            pl.when(s < NS - 1)(_zero_wait_main)
            pl.when(s == NS - 1)(_zero_wait_last)
            plsc.subcore_barrier()
            inner()
            for hh in range(1, NCHUNK // HCH):
                stage_and_prime(s * NCHUNK + hh * HCH)
                inner()

        pl.when(c == 0)(lambda: run(xlo_hbm))
        pl.when(c == 1)(lambda: run(xhi_hbm))
        plsc.subcore_barrier()

        pl.when(c == 0)(lambda: shard_copy(acc_sh, alo_hbm, s))
        pl.when(c == 1)(lambda: shard_copy(acc_sh, ahi_hbm, s))

    @functools.partial(
        pl.kernel,
        out_type=jax.ShapeDtypeStruct((HR, H), jnp.float32),
        mesh=mesh,
        compiler_params=pltpu.CompilerParams(needs_layout_passes=False),
        scratch_types=[
            pltpu.VMEM((EPT,), jnp.int32),
            pltpu.VMEM((HR, H), jnp.float32),
            pltpu.VMEM((HR,), jnp.int32),
            pltpu.VMEM((8, H), jnp.float32),
            pltpu.VMEM_SHARED((HR, H), jnp.float32),
        ],
    )
    def sc_counts(dst_hbm, zeros_hbm, inv_hbm,
                  dst_v, cnt2d, idx80, bufr, acc_sh):
        c = lax.axis_index("c")
        s = lax.axis_index("s")
        pl.when(s < HR // 8)(lambda: pltpu.sync_copy(
            zeros_hbm.at[pl.ds(s * 8, 8)], acc_sh.at[pl.ds(s * 8, 8)]))
        pltpu.sync_copy(dst_hbm.at[pl.ds(s * EPT, EPT)], dst_v)
        for k in range(HR // 16):
            idx80[pl.ds(k * 16, 16)] = lax.iota(jnp.int32, 16) + (16 * k)
        zv = jnp.zeros((16,), jnp.float32)

        def zstep(i, carry):
            r = i // 8
            k = i - r * 8
            cnt2d[r, pl.ds(k * 16, 16)] = zv
            return carry
        lax.fori_loop(0, HR * 8, zstep, 0)

        ones = jnp.ones((16,), jnp.float32)

        def estep(i, carry):
            idx = dst_v[pl.ds(i * 16, 16)]
            row = lax.shift_right_logical(idx, 7)
            col = jnp.bitwise_and(idx, 127)
            plsc.addupdate_scatter(cnt2d, [row, col], ones)
            return carry
        lax.fori_loop(0, EPT // 16, estep, 0)

        plsc.subcore_barrier()
        pltpu.sync_copy(cnt2d, acc_sh.at[idx80], add=True)
        plsc.subcore_barrier()

        def _out():
            g = c * (HR // 16) + s
            pltpu.sync_copy(acc_sh.at[pl.ds(g * 8, 8)], bufr)
            for r in range(8):
                for k in range(8):
                    v = bufr[r, pl.ds(k * 16, 16)]
                    bufr[r, pl.ds(k * 16, 16)] = 1.0 / jnp.maximum(v, 1.0)
            pltpu.sync_copy(bufr, inv_hbm.at[pl.ds(g * 8, 8)])

        pl.when(s < HR // 16)(_out)

    return sc_aggregate, sc_counts


RB = 1000
NBLK = N // RB


def _t12_body(alo, ahi, hlo, hhi, inv, wl, wr, b, g, be,
              zlo_ref, zhi_ref, y_scr, stats_acc):
    p = pl.program_id(0)
    i = pl.program_id(1)

    @pl.when(p == 0)
    def _():
        agg = jnp.concatenate([alo[...], ahi[...]], axis=1) * inv[...]
        h = jnp.concatenate([hlo[...], hhi[...]], axis=1)
        y = (jnp.dot(agg, wl[...], preferred_element_type=jnp.float32)
             + jnp.dot(h, wr[...], preferred_element_type=jnp.float32)
             + b[...])
        y_scr[pl.ds(i * RB, RB), :] = y

        @pl.when(i == 0)
        def _():
            stats_acc[...] = jnp.zeros_like(stats_acc)

        stats_acc[0:1, :] += jnp.sum(y, axis=0, keepdims=True)
        stats_acc[1:2, :] += jnp.sum(y * y, axis=0, keepdims=True)

    @pl.when(p == 1)
    def _():
        mu = stats_acc[0:1, :] * (1.0 / N)
        var = stats_acc[1:2, :] * (1.0 / N) - mu * mu
        scale = g[...] * lax.rsqrt(var + EPS)
        shift = be[...] - scale * mu
        z = jnp.maximum(y_scr[pl.ds(i * RB, RB), :] * scale + shift, 0.0)
        zlo_ref[...] = z[:, :H]
        zhi_ref[...] = z[:, H:]


def _tc_layer(alo, ahi, hlo, hhi, inv2d, wl, wr, b, g, be):
    blk = lambda p, i: (jnp.where(p == 0, i, 0), 0)
    cst = lambda p, i: (0, 0)
    return pl.pallas_call(
        _t12_body,
        grid=(2, NBLK),
        in_specs=[
            pl.BlockSpec((RB, H), blk),
            pl.BlockSpec((RB, H), blk),
            pl.BlockSpec((RB, H), blk),
            pl.BlockSpec((RB, H), blk),
            pl.BlockSpec((RB, 1), blk),
            pl.BlockSpec((D, D), cst),
            pl.BlockSpec((D, D), cst),
            pl.BlockSpec((1, D), cst),
            pl.BlockSpec((1, D), cst),
            pl.BlockSpec((1, D), cst),
        ],
        out_specs=[
            pl.BlockSpec((RB, H), lambda p, i: (i, 0)),
            pl.BlockSpec((RB, H), lambda p, i: (i, 0)),
        ],
        out_shape=[
            jax.ShapeDtypeStruct((N, H), jnp.float32),
            jax.ShapeDtypeStruct((N, H), jnp.float32),
        ],
        scratch_shapes=[pltpu.VMEM((N, D), jnp.float32),
                        pltpu.VMEM((8, D), jnp.float32)],
    )(alo, ahi, hlo, hhi, inv2d, wl, wr, b, g, be)


def _t3_body(alo, ahi, hlo, hhi, inv, wl, wr, b, batch,
             out_ref, pool_acc, cg_acc):
    i = pl.program_id(0)
    agg = jnp.concatenate([alo[...], ahi[...]], axis=1) * inv[...]
    h = jnp.concatenate([hlo[...], hhi[...]], axis=1)
    y = (jnp.dot(agg, wl[...], preferred_element_type=jnp.float32)
         + jnp.dot(h, wr[...], preferred_element_type=jnp.float32)
         + b[...])
    bb = batch[0, 0, :]
    oh = (bb[:, None] == lax.broadcasted_iota(jnp.int32, (RB, G), 1))
    oh = oh.astype(jnp.float32)

    @pl.when(i == 0)
    def _():
        pool_acc[...] = jnp.zeros_like(pool_acc)
        cg_acc[...] = jnp.zeros_like(cg_acc)

    pool_acc[...] += lax.dot_general(oh, y, (((0,), (0,)), ((), ())),
                                     preferred_element_type=jnp.float32)
    cg_acc[...] += lax.dot_general(oh, jnp.ones((RB, H), jnp.float32),
                                   (((0,), (0,)), ((), ())),
                                   preferred_element_type=jnp.float32)

    @pl.when(i == NBLK - 1)
    def _():
        out_ref[...] = pool_acc[...] / jnp.maximum(cg_acc[:, 0:1], 1.0)


def _tc_matmul_pool(alo, ahi, hlo, hhi, inv2d, wl, wr, b, batch3):
    return pl.pallas_call(
        _t3_body,
        grid=(NBLK,),
        in_specs=[
            pl.BlockSpec((RB, H), lambda i: (i, 0)),
            pl.BlockSpec((RB, H), lambda i: (i, 0)),
            pl.BlockSpec((RB, H), lambda i: (i, 0)),
            pl.BlockSpec((RB, H), lambda i: (i, 0)),
            pl.BlockSpec((RB, 1), lambda i: (i, 0)),
            pl.BlockSpec((D, D), lambda i: (0, 0)),
            pl.BlockSpec((D, D), lambda i: (0, 0)),
            pl.BlockSpec((1, D), lambda i: (0, 0)),
            pl.BlockSpec((1, 1, RB), lambda i: (i, 0, 0)),
        ],
        out_specs=pl.BlockSpec((G, D), lambda i: (0, 0)),
        out_shape=jax.ShapeDtypeStruct((G, D), jnp.float32),
        scratch_shapes=[pltpu.VMEM((G, D), jnp.float32),
                        pltpu.VMEM((G, H), jnp.float32)],
    )(alo, ahi, hlo, hhi, inv2d, wl, wr, b, batch3)


def kernel(x, edge_index, batch, W1l, W1r, b1, g1, be1,
           W2l, W2r, b2, g2, be2, W3l, W3r, b3):
    xlo = x[:, :H]
    xhi = x[:, H:]
    src2 = edge_index[0].reshape(E // CH, CH)
    dst2 = edge_index[1].reshape(E // CH, CH)
    dst1 = edge_index[1]
    zeros128 = jnp.zeros((N, H), jnp.float32)
    batch3 = batch.reshape(NBLK, 1, RB)
    b1r = b1.reshape(1, D)
    g1r = g1.reshape(1, D)
    be1r = be1.reshape(1, D)
    b2r = b2.reshape(1, D)
    g2r = g2.reshape(1, D)
    be2r = be2.reshape(1, D)
    b3r = b3.reshape(1, D)

    _sc_aggregate, _sc_counts = _sc_kernels()

    inv80 = _sc_counts(dst1, zeros128)
    inv2d = inv80.reshape(HR * H)[:N].reshape(N, 1)

    a1lo, a1hi = _sc_aggregate(xlo, xhi, src2, dst2, zeros128)
    h1lo, h1hi = _tc_layer(a1lo, a1hi, xlo, xhi, inv2d,
                           W1l, W1r, b1r, g1r, be1r)

    a2lo, a2hi = _sc_aggregate(h1lo, h1hi, src2, dst2, zeros128)
    h2lo, h2hi = _tc_layer(a2lo, a2hi, h1lo, h1hi, inv2d,
                           W2l, W2r, b2r, g2r, be2r)

    a3lo, a3hi = _sc_aggregate(h2lo, h2hi, src2, dst2, zeros128)
    return _tc_matmul_pool(a3lo, a3hi, h2lo, h2hi, inv2d,
                           W3l, W3r, b3r, batch3)


# speedup vs baseline: 1.1944x; 1.0250x over previous
import functools

import jax
import jax.numpy as jnp
from jax import lax
from jax.experimental import pallas as pl
from jax.experimental.pallas import tpu as pltpu
from jax.experimental.pallas import tpu_sc as plsc

N = 10000
E = 160000
D = 256
H = 128
G = 64
EPS = 1e-5

NC = 2
NS = 16

CH = 125
NCHUNK = (E // NS) // CH
HCH = 40
WB = 640
WB_LAST = N - WB * (NS - 1)

HR = 80
EPT = E // NS

@functools.cache
def _sc_kernels():
    mesh = plsc.VectorSubcoreMesh(core_axis_name="c", subcore_axis_name="s",
                                  num_cores=NC, num_subcores=NS)

    def shard_copy(src_ref, dst_ref, s, **kw):
        pl.when(s < NS - 1)(lambda: pltpu.sync_copy(
            src_ref.at[pl.ds(s * WB, WB)], dst_ref.at[pl.ds(s * WB, WB)], **kw))
        pl.when(s == NS - 1)(lambda: pltpu.sync_copy(
            src_ref.at[pl.ds((NS - 1) * WB, WB_LAST)],
            dst_ref.at[pl.ds((NS - 1) * WB, WB_LAST)], **kw))

    @functools.partial(
        pl.kernel,
        out_type=[jax.ShapeDtypeStruct((N, H), jnp.float32),
                  jax.ShapeDtypeStruct((N, H), jnp.float32)],
        mesh=mesh,
        scratch_types=[
            pltpu.VMEM((HCH, CH), jnp.int32),
            pltpu.VMEM((HCH, CH), jnp.int32),
            pltpu.VMEM((CH, H), jnp.float32),
            pltpu.VMEM((CH, H), jnp.float32),
            pltpu.VMEM_SHARED((N, H), jnp.float32),
            pltpu.SemaphoreType.DMA,
            pltpu.SemaphoreType.DMA,
            pltpu.SemaphoreType.DMA,
        ],
    )
    def sc_aggregate(xlo_hbm, xhi_hbm, src_hbm, dst_hbm, zeros_hbm,
                     alo_hbm, ahi_hbm,
                     src_v, dst_v, rows_v0, rows_v1, acc_sh,
                     sem0, sem1, semz):
        c = lax.axis_index("c")
        s = lax.axis_index("s")
        def _zero_start_main():
            pltpu.async_copy(zeros_hbm.at[pl.ds(s * WB, WB)],
                             acc_sh.at[pl.ds(s * WB, WB)], semz)

        def _zero_start_last():
            pltpu.async_copy(zeros_hbm.at[pl.ds((NS - 1) * WB, WB_LAST)],
                             acc_sh.at[pl.ds((NS - 1) * WB, WB_LAST)], semz)

        pl.when(s < NS - 1)(_zero_start_main)
        pl.when(s == NS - 1)(_zero_start_last)

        def run(x_hbm):
            def g_start(j, buf, sem):
                pltpu.async_copy(x_hbm.at[src_v.at[j]], buf, sem)

            def g_wait(buf, sem):
                pltpu.make_async_copy(x_hbm.at[src_v.at[0]], buf, sem).wait()

            def stage_and_prime(hbase):
                pltpu.sync_copy(src_hbm.at[pl.ds(hbase, HCH)], src_v)
                pltpu.sync_copy(dst_hbm.at[pl.ds(hbase, HCH)], dst_v)
                g_start(0, rows_v0, sem0)
                g_start(1, rows_v1, sem1)

            def inner():
                def step(jj, carry):
                    j0 = jj * 2
                    j1 = j0 + 1
                    g_wait(rows_v0, sem0)
                    pltpu.sync_copy(rows_v0, acc_sh.at[dst_v.at[j0]],
                                    add=True)
                    pl.when(j0 + 2 < HCH)(
                        lambda: g_start(j0 + 2, rows_v0, sem0))
                    g_wait(rows_v1, sem1)
                    pltpu.sync_copy(rows_v1, acc_sh.at[dst_v.at[j1]],
                                    add=True)
                    pl.when(j1 + 2 < HCH)(
                        lambda: g_start(j1 + 2, rows_v1, sem1))
                    return carry
                lax.fori_loop(0, HCH // 2, step, 0)

            stage_and_prime(s * NCHUNK)

            def _zero_wait_main():
                pltpu.make_async_copy(
                    zeros_hbm.at[pl.ds(s * WB, WB)],
                    acc_sh.at[pl.ds(s * WB, WB)], semz).wait()

            def _zero_wait_last():
                pltpu.make_async_copy(
                    zeros_hbm.at[pl.ds((NS - 1) * WB, WB_LAST)],
                    acc_sh.at[pl.ds((NS - 1) * WB, WB_LAST)], semz).wait()

            pl.when(s < NS - 1)(_zero_wait_main)
            pl.when(s == NS - 1)(_zero_wait_last)
            plsc.subcore_barrier()
            inner()
            for hh in range(1, NCHUNK // HCH):
                stage_and_prime(s * NCHUNK + hh * HCH)
                inner()

        pl.when(c == 0)(lambda: run(xlo_hbm))
        pl.when(c == 1)(lambda: run(xhi_hbm))
        plsc.subcore_barrier()

        pl.when(c == 0)(lambda: shard_copy(acc_sh, alo_hbm, s))
        pl.when(c == 1)(lambda: shard_copy(acc_sh, ahi_hbm, s))

    @functools.partial(
        pl.kernel,
        out_type=jax.ShapeDtypeStruct((HR, H), jnp.float32),
        mesh=mesh,
        compiler_params=pltpu.CompilerParams(needs_layout_passes=False),
        scratch_types=[
            pltpu.VMEM((EPT,), jnp.int32),
            pltpu.VMEM((HR, H), jnp.float32),
            pltpu.VMEM((HR,), jnp.int32),
            pltpu.VMEM((8, H), jnp.float32),
            pltpu.VMEM_SHARED((HR, H), jnp.float32),
        ],
    )
    def sc_counts(dst_hbm, zeros_hbm, inv_hbm,
                  dst_v, cnt2d, idx80, bufr, acc_sh):
        c = lax.axis_index("c")
        s = lax.axis_index("s")
        pl.when(s < HR // 8)(lambda: pltpu.sync_copy(
            zeros_hbm.at[pl.ds(s * 8, 8)], acc_sh.at[pl.ds(s * 8, 8)]))
        pltpu.sync_copy(dst_hbm.at[pl.ds(s * EPT, EPT)], dst_v)
        for k in range(HR // 16):
            idx80[pl.ds(k * 16, 16)] = lax.iota(jnp.int32, 16) + (16 * k)
        zv = jnp.zeros((16,), jnp.float32)

        def zstep(i, carry):
            r = i // 8
            k = i - r * 8
            cnt2d[r, pl.ds(k * 16, 16)] = zv
            return carry
        lax.fori_loop(0, HR * 8, zstep, 0)

        ones = jnp.ones((16,), jnp.float32)

        def estep(i, carry):
            idx = dst_v[pl.ds(i * 16, 16)]
            row = lax.shift_right_logical(idx, 7)
            col = jnp.bitwise_and(idx, 127)
            plsc.addupdate_scatter(cnt2d, [row, col], ones)
            return carry
        lax.fori_loop(0, EPT // 16, estep, 0)

        plsc.subcore_barrier()
        pltpu.sync_copy(cnt2d, acc_sh.at[idx80], add=True)
        plsc.subcore_barrier()

        def _out():
            g = c * (HR // 16) + s
            pltpu.sync_copy(acc_sh.at[pl.ds(g * 8, 8)], bufr)
            for r in range(8):
                for k in range(8):
                    v = bufr[r, pl.ds(k * 16, 16)]
                    bufr[r, pl.ds(k * 16, 16)] = 1.0 / jnp.maximum(v, 1.0)
            pltpu.sync_copy(bufr, inv_hbm.at[pl.ds(g * 8, 8)])

        pl.when(s < HR // 16)(_out)

    return sc_aggregate, sc_counts


RB = 2000
NBLK = N // RB


def _t12_body(alo, ahi, hlo, hhi, inv, wl, wr, b, g, be,
              zlo_ref, zhi_ref, y_scr, stats_acc):
    p = pl.program_id(0)
    i = pl.program_id(1)

    @pl.when(p == 0)
    def _():
        agg = jnp.concatenate([alo[...], ahi[...]], axis=1) * inv[...]
        h = jnp.concatenate([hlo[...], hhi[...]], axis=1)
        y = (jnp.dot(agg, wl[...], preferred_element_type=jnp.float32)
             + jnp.dot(h, wr[...], preferred_element_type=jnp.float32)
             + b[...])
        y_scr[pl.ds(i * RB, RB), :] = y

        @pl.when(i == 0)
        def _():
            stats_acc[...] = jnp.zeros_like(stats_acc)

        stats_acc[0:1, :] += jnp.sum(y, axis=0, keepdims=True)
        stats_acc[1:2, :] += jnp.sum(y * y, axis=0, keepdims=True)

    @pl.when(p == 1)
    def _():
        mu = stats_acc[0:1, :] * (1.0 / N)
        var = stats_acc[1:2, :] * (1.0 / N) - mu * mu
        scale = g[...] * lax.rsqrt(var + EPS)
        shift = be[...] - scale * mu
        z = jnp.maximum(y_scr[pl.ds(i * RB, RB), :] * scale + shift, 0.0)
        zlo_ref[...] = z[:, :H]
        zhi_ref[...] = z[:, H:]


def _tc_layer(alo, ahi, hlo, hhi, inv2d, wl, wr, b, g, be):
    blk = lambda p, i: (jnp.where(p == 0, i, 0), 0)
    cst = lambda p, i: (0, 0)
    return pl.pallas_call(
        _t12_body,
        grid=(2, NBLK),
        in_specs=[
            pl.BlockSpec((RB, H), blk),
            pl.BlockSpec((RB, H), blk),
            pl.BlockSpec((RB, H), blk),
            pl.BlockSpec((RB, H), blk),
            pl.BlockSpec((RB, 1), blk),
            pl.BlockSpec((D, D), cst),
            pl.BlockSpec((D, D), cst),
            pl.BlockSpec((1, D), cst),
            pl.BlockSpec((1, D), cst),
            pl.BlockSpec((1, D), cst),
        ],
        out_specs=[
            pl.BlockSpec((RB, H), lambda p, i: (i, 0)),
            pl.BlockSpec((RB, H), lambda p, i: (i, 0)),
        ],
        out_shape=[
            jax.ShapeDtypeStruct((N, H), jnp.float32),
            jax.ShapeDtypeStruct((N, H), jnp.float32),
        ],
        scratch_shapes=[pltpu.VMEM((N, D), jnp.float32),
                        pltpu.VMEM((8, D), jnp.float32)],
    )(alo, ahi, hlo, hhi, inv2d, wl, wr, b, g, be)


def _t3_body(alo, ahi, hlo, hhi, inv, wl, wr, b, batch,
             out_ref, pool_acc, cg_acc):
    i = pl.program_id(0)
    agg = jnp.concatenate([alo[...], ahi[...]], axis=1) * inv[...]
    h = jnp.concatenate([hlo[...], hhi[...]], axis=1)
    y = (jnp.dot(agg, wl[...], preferred_element_type=jnp.float32)
         + jnp.dot(h, wr[...], preferred_element_type=jnp.float32)
         + b[...])
    bb = batch[0, 0, :]
    oh = (bb[:, None] == lax.broadcasted_iota(jnp.int32, (RB, G), 1))
    oh = oh.astype(jnp.float32)

    @pl.when(i == 0)
    def _():
        pool_acc[...] = jnp.zeros_like(pool_acc)
        cg_acc[...] = jnp.zeros_like(cg_acc)

    pool_acc[...] += lax.dot_general(oh, y, (((0,), (0,)), ((), ())),
                                     preferred_element_type=jnp.float32)
    cg_acc[...] += lax.dot_general(oh, jnp.ones((RB, H), jnp.float32),
                                   (((0,), (0,)), ((), ())),
                                   preferred_element_type=jnp.float32)

    @pl.when(i == NBLK - 1)
    def _():
        out_ref[...] = pool_acc[...] / jnp.maximum(cg_acc[:, 0:1], 1.0)


def _tc_matmul_pool(alo, ahi, hlo, hhi, inv2d, wl, wr, b, batch3):
    return pl.pallas_call(
        _t3_body,
        grid=(NBLK,),
        in_specs=[
            pl.BlockSpec((RB, H), lambda i: (i, 0)),
            pl.BlockSpec((RB, H), lambda i: (i, 0)),
            pl.BlockSpec((RB, H), lambda i: (i, 0)),
            pl.BlockSpec((RB, H), lambda i: (i, 0)),
            pl.BlockSpec((RB, 1), lambda i: (i, 0)),
            pl.BlockSpec((D, D), lambda i: (0, 0)),
            pl.BlockSpec((D, D), lambda i: (0, 0)),
            pl.BlockSpec((1, D), lambda i: (0, 0)),
            pl.BlockSpec((1, 1, RB), lambda i: (i, 0, 0)),
        ],
        out_specs=pl.BlockSpec((G, D), lambda i: (0, 0)),
        out_shape=jax.ShapeDtypeStruct((G, D), jnp.float32),
        scratch_shapes=[pltpu.VMEM((G, D), jnp.float32),
                        pltpu.VMEM((G, H), jnp.float32)],
    )(alo, ahi, hlo, hhi, inv2d, wl, wr, b, batch3)


def kernel(x, edge_index, batch, W1l, W1r, b1, g1, be1,
           W2l, W2r, b2, g2, be2, W3l, W3r, b3):
    xlo = x[:, :H]
    xhi = x[:, H:]
    src2 = edge_index[0].reshape(E // CH, CH)
    dst2 = edge_index[1].reshape(E // CH, CH)
    dst1 = edge_index[1]
    zeros128 = jnp.zeros((N, H), jnp.float32)
    batch3 = batch.reshape(NBLK, 1, RB)
    b1r = b1.reshape(1, D)
    g1r = g1.reshape(1, D)
    be1r = be1.reshape(1, D)
    b2r = b2.reshape(1, D)
    g2r = g2.reshape(1, D)
    be2r = be2.reshape(1, D)
    b3r = b3.reshape(1, D)

    _sc_aggregate, _sc_counts = _sc_kernels()

    inv80 = _sc_counts(dst1, zeros128)
    inv2d = inv80.reshape(HR * H)[:N].reshape(N, 1)

    a1lo, a1hi = _sc_aggregate(xlo, xhi, src2, dst2, zeros128)
    h1lo, h1hi = _tc_layer(a1lo, a1hi, xlo, xhi, inv2d,
                           W1l, W1r, b1r, g1r, be1r)

    a2lo, a2hi = _sc_aggregate(h1lo, h1hi, src2, dst2, zeros128)
    h2lo, h2hi = _tc_layer(a2lo, a2hi, h1lo, h1hi, inv2d,
                           W2l, W2r, b2r, g2r, be2r)

    a3lo, a3hi = _sc_aggregate(h2lo, h2hi, src2, dst2, zeros128)
    return _tc_matmul_pool(a3lo, a3hi, h2lo, h2hi, inv2d,
                           W3l, W3r, b3r, batch3)
